# baseline (speedup 1.0000x reference)
import jax
import jax.numpy as jnp
from jax.experimental import pallas as pl
from jax.experimental.pallas import tpu as pltpu

BT = 32
_S = 256
_NB = 8
_BS = 32
_INV_SQRT_D = 1.0 / 16.0

_HI = jax.lax.Precision.HIGHEST
_BF = jnp.bfloat16
_F32 = jnp.float32


def _mmT_hi(a, b):
    return jax.lax.dot_general(a, b, (((1,), (1,)), ((), ())),
                               precision=_HI, preferred_element_type=_F32)


def _mmT_bf(a, b):
    return jax.lax.dot_general(a.astype(_BF), b.astype(_BF),
                               (((1,), (1,)), ((), ())),
                               preferred_element_type=_F32)


def _fused(obs_ref, msg_ref,
           W1_ref, b1_ref, W2_ref, b2_ref,
           Wq_ref, bq_ref, Wk_ref, bk_ref,
           Wv_ref, bv_ref, Wo_ref, bo_ref,
           W3_ref, b3_ref, W4_ref, b4_ref,
           out_ref):
    obs = obs_ref[...]

    x1 = jnp.maximum(_mmT_bf(obs, W1_ref[...]) + b1_ref[...], 0.0)
    xe = jnp.maximum(_mmT_bf(x1, W2_ref[...]) + b2_ref[...], 0.0)

    q0 = _mmT_bf(xe, Wq_ref[...]) + bq_ref[...]
    k_self = _mmT_bf(xe, Wk_ref[...]) + bk_ref[...]
    q0_bf = q0.astype(_BF)
    Wk = Wk_ref[...]
    bk = bk_ref[...]

    rows = []
    for b in range(BT):
        kb = _mmT_bf(msg_ref[b], Wk) + bk
        rows.append(jax.lax.dot_general(
            q0_bf[b:b + 1], kb.astype(_BF), (((1,), (1,)), ((), ())),
            preferred_element_type=_F32))
    s_msg = jnp.concatenate(rows, axis=0)
    s_self = jnp.sum(q0_bf.astype(_F32) * k_self.astype(_BF).astype(_F32),
                     axis=-1, keepdims=True)
    s = jnp.concatenate([s_self, s_msg], axis=1) * _INV_SQRT_D

    jj = jax.lax.broadcasted_iota(jnp.int32, (_S, _NB), 0)
    ii = jax.lax.broadcasted_iota(jnp.int32, (_S, _NB), 1)
    eblk = (jj // _BS == ii).astype(_F32)
    bs = jax.lax.dot_general(s, eblk, (((1,), (0,)), ((), ())),
                             precision=_HI,
                             preferred_element_type=_F32) * (1.0 / _BS)
    m1 = jnp.max(bs, axis=-1, keepdims=True)
    ismax = bs >= m1
    nmax = jnp.sum(ismax.astype(_F32), axis=-1, keepdims=True)
    second = jnp.max(jnp.where(ismax, -jnp.inf, bs), axis=-1, keepdims=True)
    thresh = jnp.where(nmax >= 2.0, m1, second)

    bmask = (bs >= thresh).astype(_F32)
    mask256 = jax.lax.dot_general(bmask, eblk.T, (((1,), (0,)), ((), ())),
                                  preferred_element_type=_F32)
    sm = jnp.where(mask256 > 0.5, s, _F32(-1e9))

    rowmax = jnp.max(sm, axis=-1, keepdims=True)
    e = jnp.exp(sm - rowmax)
    attn = e / jnp.sum(e, axis=-1, keepdims=True)

    a_msg = attn[:, 1:]
    wrows = []
    for b in range(BT):
        wrows.append(jax.lax.dot_general(
            a_msg[b:b + 1], msg_ref[b], (((1,), (0,)), ((), ())),
            preferred_element_type=_F32))
    wsum = jnp.concatenate(wrows, axis=0) + attn[:, 0:1] * xe

    o = _mmT_hi(wsum, Wv_ref[...]) + bv_ref[...]
    comm = _mmT_hi(o, Wo_ref[...]) + bo_ref[...]
    x = xe + comm

    h = jnp.maximum(_mmT_hi(x, W3_ref[...]) + b3_ref[...], 0.0)
    out_ref[...] = _mmT_hi(h, W4_ref[...]) + b4_ref[...]


def kernel(local_obs, messages, W1, b1, W2, b2, Wq, bq, Wk, bk, Wv, bv,
           Wo, bo, W3, b3, W4, b4):
    B = local_obs.shape[0]
    grid = (B // BT,)

    def row2(n):
        return jnp.reshape(n, (1, -1))

    def wspec(shape):
        nd = len(shape)
        return pl.BlockSpec(shape, lambda i: (0,) * nd)

    weight_args = (W1, row2(b1), W2, row2(b2),
                   Wq, row2(bq), Wk, row2(bk),
                   Wv, row2(bv), Wo, row2(bo),
                   W3, row2(b3), W4, row2(b4))
    weight_specs = [wspec(w.shape) for w in weight_args]

    return pl.pallas_call(
        _fused,
        grid=grid,
        in_specs=[
            pl.BlockSpec((BT, 512), lambda i: (i, 0)),
            pl.BlockSpec((BT, 255, 256), lambda i: (i, 0, 0)),
            *weight_specs,
        ],
        out_specs=pl.BlockSpec((BT, 128), lambda i: (i, 0)),
        out_shape=jax.ShapeDtypeStruct((B, 128), jnp.float32),
        compiler_params=pltpu.CompilerParams(
            vmem_limit_bytes=110 * 1024 * 1024),
    )(local_obs, messages, *weight_args)


# speedup vs baseline: 1.5491x; 1.5491x over previous
import jax
import jax.numpy as jnp
from jax.experimental import pallas as pl
from jax.experimental.pallas import tpu as pltpu

BT = 32
_NM = 255
_S = 256
_NB = 8
_BS = 32
_INV_SQRT_D = 1.0 / 16.0

_HI = jax.lax.Precision.HIGHEST
_BF = jnp.bfloat16
_F32 = jnp.float32


def _mmT_hi(a, b):
    return jax.lax.dot_general(a, b, (((1,), (1,)), ((), ())),
                               precision=_HI, preferred_element_type=_F32)


def _mmT_bf(a, b):
    return jax.lax.dot_general(a.astype(_BF), b.astype(_BF),
                               (((1,), (1,)), ((), ())),
                               preferred_element_type=_F32)


def _fused(obs_ref, msg_ref,
           W1_ref, b1_ref, W2_ref, b2_ref,
           Wq_ref, bq_ref, Wk_ref, bk_ref,
           Wv_ref, bv_ref, Wo_ref, bo_ref,
           W3_ref, b3_ref, W4_ref, b4_ref,
           out_ref):
    obs = obs_ref[...]
    mflat = msg_ref[...].reshape(_NM * BT, 256)

    x1 = jnp.maximum(_mmT_bf(obs, W1_ref[...]) + b1_ref[...], 0.0)
    xe = jnp.maximum(_mmT_bf(x1, W2_ref[...]) + b2_ref[...], 0.0)

    q0 = _mmT_bf(xe, Wq_ref[...]) + bq_ref[...]
    k_self = _mmT_bf(xe, Wk_ref[...]) + bk_ref[...]

    kflat = _mmT_bf(mflat, Wk_ref[...]) + bk_ref[...]

    s_all = _mmT_bf(kflat, q0)
    s3 = s_all.reshape(_NM, BT, BT)
    bi = jax.lax.broadcasted_iota(jnp.int32, (1, BT, BT), 1)
    ci = jax.lax.broadcasted_iota(jnp.int32, (1, BT, BT), 2)
    eye = (bi == ci).astype(_F32)
    s_t = jnp.sum(s3 * eye, axis=1)
    s_msg = s_t.T
    s_self = jnp.sum(q0.astype(_BF).astype(_F32)
                     * k_self.astype(_BF).astype(_F32),
                     axis=-1, keepdims=True)
    s = jnp.concatenate([s_self, s_msg], axis=1) * _INV_SQRT_D

    jj = jax.lax.broadcasted_iota(jnp.int32, (_S, _NB), 0)
    ii = jax.lax.broadcasted_iota(jnp.int32, (_S, _NB), 1)
    eblk = (jj // _BS == ii).astype(_F32)
    bs = jax.lax.dot_general(s, eblk, (((1,), (0,)), ((), ())),
                             precision=_HI,
                             preferred_element_type=_F32) * (1.0 / _BS)
    m1 = jnp.max(bs, axis=-1, keepdims=True)
    ismax = bs >= m1
    nmax = jnp.sum(ismax.astype(_F32), axis=-1, keepdims=True)
    second = jnp.max(jnp.where(ismax, -jnp.inf, bs), axis=-1, keepdims=True)
    thresh = jnp.where(nmax >= 2.0, m1, second)

    bmask = (bs >= thresh).astype(_F32)
    mask256 = jax.lax.dot_general(bmask, eblk.T, (((1,), (0,)), ((), ())),
                                  preferred_element_type=_F32)
    sm = jnp.where(mask256 > 0.5, s, _F32(-1e9))

    rowmax = jnp.max(sm, axis=-1, keepdims=True)
    e = jnp.exp(sm - rowmax)
    attn = e / jnp.sum(e, axis=-1, keepdims=True)

    a_t = attn[:, 1:].T
    at_flat = (a_t[:, :, None] * eye[0][None]).reshape(_NM * BT, BT)
    wsum_t = jax.lax.dot_general(
        mflat, at_flat, (((0,), (0,)), ((), ())),
        preferred_element_type=_F32)
    wsum = wsum_t.T + attn[:, 0:1] * xe

    o = _mmT_hi(wsum, Wv_ref[...]) + bv_ref[...]
    comm = _mmT_hi(o, Wo_ref[...]) + bo_ref[...]
    x = xe + comm

    h = jnp.maximum(_mmT_hi(x, W3_ref[...]) + b3_ref[...], 0.0)
    out_ref[...] = _mmT_hi(h, W4_ref[...]) + b4_ref[...]


def kernel(local_obs, messages, W1, b1, W2, b2, Wq, bq, Wk, bk, Wv, bv,
           Wo, bo, W3, b3, W4, b4):
    B = local_obs.shape[0]
    grid = (B // BT,)

    mt = jnp.transpose(messages, (1, 0, 2))

    def row2(n):
        return jnp.reshape(n, (1, -1))

    def wspec(shape):
        nd = len(shape)
        return pl.BlockSpec(shape, lambda i: (0,) * nd)

    weight_args = (W1, row2(b1), W2, row2(b2),
                   Wq, row2(bq), Wk, row2(bk),
                   Wv, row2(bv), Wo, row2(bo),
                   W3, row2(b3), W4, row2(b4))
    weight_specs = [wspec(w.shape) for w in weight_args]

    return pl.pallas_call(
        _fused,
        grid=grid,
        in_specs=[
            pl.BlockSpec((BT, 512), lambda i: (i, 0)),
            pl.BlockSpec((_NM, BT, 256), lambda i: (0, i, 0)),
            *weight_specs,
        ],
        out_specs=pl.BlockSpec((BT, 128), lambda i: (i, 0)),
        out_shape=jax.ShapeDtypeStruct((B, 128), jnp.float32),
        compiler_params=pltpu.CompilerParams(
            vmem_limit_bytes=110 * 1024 * 1024),
    )(local_obs, mt, *weight_args)
